# baseline (speedup 1.0000x reference)
import functools

import jax
import jax.numpy as jnp
from jax.experimental import pallas as pl
from jax.experimental.pallas import tpu as pltpu

N_GRAPHS_CONST = 64


def _fused_mlp_body(x_ref, w1_ref, b1_ref, g_ref, be_ref, w2_ref, b2_ref, o_ref):
    h = jnp.dot(x_ref[...], w1_ref[...], preferred_element_type=jnp.float32)
    h = h + b1_ref[...]
    h = jnp.where(h >= 0, h, 0.01 * h)
    mu = jnp.mean(h, axis=-1, keepdims=True)
    var = jnp.mean((h - mu) ** 2, axis=-1, keepdims=True)
    h = (h - mu) * jax.lax.rsqrt(var + 1e-5) * g_ref[...] + be_ref[...]
    o_ref[...] = jnp.dot(h, w2_ref[...], preferred_element_type=jnp.float32) + b2_ref[...]


def _mlp(p, x, block=512):
    n, din = x.shape
    dh = p["w1"].shape[1]
    dout = p["w2"].shape[1]
    block = min(block, n)
    grid = (pl.cdiv(n, block),)
    full = lambda shape: pl.BlockSpec(shape, lambda i: (0, 0))
    return pl.pallas_call(
        _fused_mlp_body,
        grid=grid,
        in_specs=[
            pl.BlockSpec((block, din), lambda i: (i, 0)),
            full((din, dh)),
            full((1, dh)),
            full((1, dh)),
            full((1, dh)),
            full((dh, dout)),
            full((1, dout)),
        ],
        out_specs=pl.BlockSpec((block, dout), lambda i: (i, 0)),
        out_shape=jax.ShapeDtypeStruct((n, dout), jnp.float32),
    )(
        x,
        p["w1"],
        p["b1"].reshape(1, -1),
        p["g"].reshape(1, -1),
        p["be"].reshape(1, -1),
        p["w2"],
        p["b2"].reshape(1, -1),
    )


def _scatter_mean(data, idx, n):
    s = jax.ops.segment_sum(data, idx, num_segments=n)
    c = jax.ops.segment_sum(jnp.ones((data.shape[0], 1), data.dtype), idx, num_segments=n)
    return s / jnp.maximum(c, 1.0)


def kernel(x, edge_attr, params, edge_index, batch):
    row, col = edge_index[0], edge_index[1]
    n = x.shape[0]
    p = params["l1"]
    e = _mlp(p["edge"], jnp.concatenate([x[row], x[col], edge_attr], axis=1))
    h = _mlp(p["node1"], jnp.concatenate([x[row], e], axis=1))
    agg = _scatter_mean(h, col, n)
    xn = _mlp(p["node2"], jnp.concatenate([x, agg], axis=1))
    u = _mlp(p["glob"], _scatter_mean(xn, batch, N_GRAPHS_CONST))
    x_cur, ea = xn, e
    for name in ("l2", "l3"):
        p = params[name]
        e = _mlp(p["edge"], jnp.concatenate([x_cur[row], x_cur[col], ea, u[batch[row]]], axis=1))
        h = _mlp(p["node1"], jnp.concatenate([x_cur[row], e], axis=1))
        agg = _scatter_mean(h, col, n)
        x_cur = _mlp(p["node2"], jnp.concatenate([x_cur, agg, u[batch]], axis=1))
        u = _mlp(p["glob"], jnp.concatenate([u, _scatter_mean(x_cur, batch, N_GRAPHS_CONST)], axis=1))
        ea = e
    return u


# speedup vs baseline: 1.5313x; 1.5313x over previous
import functools

import jax
import jax.numpy as jnp
from jax.experimental import pallas as pl
from jax.experimental.pallas import tpu as pltpu

NG = 64
EB = 640
NB = 400


def _act_ln(h, g, be):
    h = jnp.where(h >= 0, h, 0.01 * h)
    mu = jnp.mean(h, axis=-1, keepdims=True)
    var = jnp.mean((h - mu) ** 2, axis=-1, keepdims=True)
    return (h - mu) * jax.lax.rsqrt(var + 1e-5) * g + be


def _dot(a, b):
    return jnp.dot(a, b, preferred_element_type=jnp.float32)


def _k1_body(has_u, xr_ref, xc_ref, ea_ref, brow_ref, ub_ref,
             wa, wb, wc, b1e, ge, bee, w2e, b2e,
             wna, wnb, b1n, gn, ben, w2n, b2n,
             e_ref, h_ref):
    xr = xr_ref[...]
    pre = _dot(xr, wa[...]) + _dot(xc_ref[...], wb[...]) + _dot(ea_ref[...], wc[...]) + b1e[...]
    if has_u:
        iot = jax.lax.broadcasted_iota(jnp.int32, (xr.shape[0], NG), 1)
        oh = (brow_ref[...] == iot).astype(jnp.float32)
        pre = pre + _dot(oh, ub_ref[...])
    e = _dot(_act_ln(pre, ge[...], bee[...]), w2e[...]) + b2e[...]
    e_ref[...] = e
    preh = _dot(xr, wna[...]) + _dot(e, wnb[...]) + b1n[...]
    h_ref[...] = _dot(_act_ln(preh, gn[...], ben[...]), w2n[...]) + b2n[...]


def _edge_node1(pe, pn, xr, xc, ea, brow, ub):
    E, dx = xr.shape
    de = ea.shape[1]
    H = pe["w2"].shape[0]
    has_u = ub is not None
    wa = pe["w1"][:dx]
    wb = pe["w1"][dx:2 * dx]
    wc = pe["w1"][2 * dx:2 * dx + de]
    wna = pn["w1"][:dx]
    wnb = pn["w1"][dx:dx + H]
    r1 = lambda a: a.reshape(1, -1)
    full = lambda shape: pl.BlockSpec(shape, lambda i: (0,) * len(shape))
    grid = (pl.cdiv(E, EB),)
    if not has_u:
        brow = jnp.zeros((E, 1), jnp.int32)
        ub = jnp.zeros((NG, H), jnp.float32)
    args = [xr, xc, ea, brow, ub,
            wa, wb, wc, r1(pe["b1"]), r1(pe["g"]), r1(pe["be"]), pe["w2"], r1(pe["b2"]),
            wna, wnb, r1(pn["b1"]), r1(pn["g"]), r1(pn["be"]), pn["w2"], r1(pn["b2"])]
    in_specs = [
        pl.BlockSpec((EB, dx), lambda i: (i, 0)),
        pl.BlockSpec((EB, dx), lambda i: (i, 0)),
        pl.BlockSpec((EB, de), lambda i: (i, 0)),
        pl.BlockSpec((EB, 1), lambda i: (i, 0)),
        full((NG, H)),
        full((dx, H)), full((dx, H)), full((de, H)), full((1, H)), full((1, H)),
        full((1, H)), full((H, H)), full((1, H)),
        full((dx, H)), full((H, H)), full((1, H)), full((1, H)), full((1, H)),
        full((H, H)), full((1, H)),
    ]
    return pl.pallas_call(
        functools.partial(_k1_body, has_u),
        grid=grid,
        in_specs=in_specs,
        out_specs=[pl.BlockSpec((EB, H), lambda i: (i, 0)),
                   pl.BlockSpec((EB, H), lambda i: (i, 0))],
        out_shape=[jax.ShapeDtypeStruct((E, H), jnp.float32),
                   jax.ShapeDtypeStruct((E, H), jnp.float32)],
    )(*args)


def _k3_body(has_u, x_ref, ssum_ref, cnt_ref, batch_ref, ub2_ref,
             wa, wb, b1, g, be, w2, b2,
             xn_ref, gsum_ref, gcnt_ref):
    xv = x_ref[...]
    agg = ssum_ref[...] * (1.0 / jnp.maximum(cnt_ref[...], 1.0))
    pre = _dot(xv, wa[...]) + _dot(agg, wb[...]) + b1[...]
    iot = jax.lax.broadcasted_iota(jnp.int32, (xv.shape[0], NG), 1)
    oh = (batch_ref[...] == iot).astype(jnp.float32)
    if has_u:
        pre = pre + _dot(oh, ub2_ref[...])
    xn = _dot(_act_ln(pre, g[...], be[...]), w2[...]) + b2[...]
    xn_ref[...] = xn
    dn = (((0,), (0,)), ((), ()))
    psum = jax.lax.dot_general(oh, xn, dn, preferred_element_type=jnp.float32)
    pcnt = jax.lax.dot_general(oh, jnp.ones_like(xn), dn, preferred_element_type=jnp.float32)

    @pl.when(pl.program_id(0) == 0)
    def _():
        gsum_ref[...] = jnp.zeros_like(gsum_ref)
        gcnt_ref[...] = jnp.zeros_like(gcnt_ref)

    gsum_ref[...] += psum
    gcnt_ref[...] += pcnt


def _node2(pn2, x_cur, ssum, cnt, batch_c, ub2):
    n, dx = x_cur.shape
    H = pn2["w2"].shape[0]
    has_u = ub2 is not None
    wa = pn2["w1"][:dx]
    wb = pn2["w1"][dx:dx + H]
    r1 = lambda a: a.reshape(1, -1)
    full = lambda shape: pl.BlockSpec(shape, lambda i: (0,) * len(shape))
    if not has_u:
        ub2 = jnp.zeros((NG, H), jnp.float32)
    grid = (pl.cdiv(n, NB),)
    args = [x_cur, ssum, cnt, batch_c, ub2,
            wa, wb, r1(pn2["b1"]), r1(pn2["g"]), r1(pn2["be"]), pn2["w2"], r1(pn2["b2"])]
    in_specs = [
        pl.BlockSpec((NB, dx), lambda i: (i, 0)),
        pl.BlockSpec((NB, H), lambda i: (i, 0)),
        pl.BlockSpec((NB, 1), lambda i: (i, 0)),
        pl.BlockSpec((NB, 1), lambda i: (i, 0)),
        full((NG, H)),
        full((dx, H)), full((H, H)), full((1, H)), full((1, H)), full((1, H)),
        full((H, H)), full((1, H)),
    ]
    return pl.pallas_call(
        functools.partial(_k3_body, has_u),
        grid=grid,
        in_specs=in_specs,
        out_specs=[pl.BlockSpec((NB, H), lambda i: (i, 0)),
                   full((NG, H)), full((NG, H))],
        out_shape=[jax.ShapeDtypeStruct((n, H), jnp.float32),
                   jax.ShapeDtypeStruct((NG, H), jnp.float32),
                   jax.ShapeDtypeStruct((NG, H), jnp.float32)],
    )(*args)


def _k4_body(has_u, u_ref, gsum_ref, gcnt_ref,
             wu, wm, b1, g, be, w2, b2, wde, wd2,
             uo_ref, ube_ref, ub2_ref):
    mean = gsum_ref[...] * (1.0 / jnp.maximum(gcnt_ref[...], 1.0))
    pre = _dot(mean, wm[...]) + b1[...]
    if has_u:
        pre = pre + _dot(u_ref[...], wu[...])
    uo = _dot(_act_ln(pre, g[...], be[...]), w2[...]) + b2[...]
    uo_ref[...] = uo
    ube_ref[...] = _dot(uo, wde[...])
    ub2_ref[...] = _dot(uo, wd2[...])


def _glob(pg, u, gsum, gcnt, wde, wd2):
    H = gsum.shape[1]
    GH = pg["w2"].shape[1]
    has_u = u is not None
    if has_u:
        wu = pg["w1"][:GH]
        wm = pg["w1"][GH:GH + H]
    else:
        u = jnp.zeros((NG, GH), jnp.float32)
        wu = jnp.zeros((GH, pg["w1"].shape[1]), jnp.float32)
        wm = pg["w1"]
    HH = wde.shape[1]
    r1 = lambda a: a.reshape(1, -1)
    return pl.pallas_call(
        functools.partial(_k4_body, has_u),
        out_shape=[jax.ShapeDtypeStruct((NG, GH), jnp.float32),
                   jax.ShapeDtypeStruct((NG, HH), jnp.float32),
                   jax.ShapeDtypeStruct((NG, HH), jnp.float32)],
    )(u, gsum, gcnt, wu, wm, r1(pg["b1"]), r1(pg["g"]), r1(pg["be"]),
      pg["w2"], r1(pg["b2"]), wde, wd2)


def kernel(x, edge_attr, params, edge_index, batch):
    row, col = edge_index[0], edge_index[1]
    n = x.shape[0]
    E = row.shape[0]
    H = 256
    batch_c = batch.reshape(n, 1)
    brow = batch[row].reshape(E, 1)
    cnt = jax.ops.segment_sum(jnp.ones((E, 1), jnp.float32), col, num_segments=n)

    p = params["l1"]
    xr, xc = x[row], x[col]
    e, h = _edge_node1(p["edge"], p["node1"], xr, xc, edge_attr, None, None)
    ssum = jax.ops.segment_sum(h, col, num_segments=n)
    xn, gsum, gcnt = _node2(p["node2"], x, ssum, cnt, batch_c, None)
    pn = params["l2"]
    u, ube, ub2 = _glob(p["glob"], None, gsum, gcnt,
                        pn["edge"]["w1"][3 * H:], pn["node2"]["w1"][2 * H:])
    x_cur, ea = xn, e

    for name, nxt in (("l2", "l3"), ("l3", None)):
        p = params[name]
        xr, xc = x_cur[row], x_cur[col]
        e, h = _edge_node1(p["edge"], p["node1"], xr, xc, ea, brow, ube)
        ssum = jax.ops.segment_sum(h, col, num_segments=n)
        xn, gsum, gcnt = _node2(p["node2"], x_cur, ssum, cnt, batch_c, ub2)
        if nxt is None:
            wde = jnp.zeros((params[name]["glob"]["w2"].shape[1], H), jnp.float32)
            wd2 = wde
        else:
            pn = params[nxt]
            wde = pn["edge"]["w1"][3 * H:]
            wd2 = pn["node2"]["w1"][2 * H:]
        u, ube, ub2 = _glob(p["glob"], u, gsum, gcnt, wde, wd2)
        x_cur, ea = xn, e
    return u


# speedup vs baseline: 1.8648x; 1.2178x over previous
import functools

import jax
import jax.numpy as jnp
from jax import lax
from jax.experimental import pallas as pl
from jax.experimental.pallas import tpu as pltpu
from jax.experimental.pallas import tpu_sc as plsc

NG = 64
EB = 640
NB = 400
NC, NS = 2, 16
NW = NC * NS
GC = 128
NPAD = 10240


def _sc_mesh():
    return plsc.VectorSubcoreMesh(
        core_axis_name="c", subcore_axis_name="s", num_cores=NC, num_subcores=NS)


def _sc_gather2(table, idxr2, idxc2):
    n, D = table.shape
    nch = idxr2.shape[0]
    E = nch * GC
    kmax = pl.cdiv(nch, NW)

    def body(tab, ir, ic, xr_o, xc_o, iv, buf, sem):
        w = lax.axis_index("s") * NC + lax.axis_index("c")

        def make_loop(isrc, out):
            def lp(k, car):
                g = w + NW * k

                @pl.when(g < nch)
                def _():
                    pltpu.sync_copy(isrc.at[g], iv)
                    pltpu.async_copy(tab.at[iv], buf, sem).wait()
                    pltpu.sync_copy(buf, out.at[pl.ds(g * GC, GC)])
                return car
            return lp

        lax.fori_loop(0, kmax, make_loop(ir, xr_o), 0)
        lax.fori_loop(0, kmax, make_loop(ic, xc_o), 0)

    return pl.kernel(
        body,
        out_type=[jax.ShapeDtypeStruct((E, D), jnp.float32),
                  jax.ShapeDtypeStruct((E, D), jnp.float32)],
        mesh=_sc_mesh(),
        scratch_types=[pltpu.VMEM((GC,), jnp.int32),
                       pltpu.VMEM((GC, D), jnp.float32),
                       pltpu.SemaphoreType.DMA],
    )(table, idxr2, idxc2)


SGC = 64


def _sc_scatter(h, idx2):
    E, D = h.shape
    half = D // NC
    rows_pt = NPAD // NS
    nch = idx2.shape[0]
    kmax = pl.cdiv(nch, NS)

    def body(h_ref, ic, ssum_o, cnt_o, acc_sh, cnt_sh, iv2, hbuf, zbuf16,
             ones16):
        c = lax.axis_index("c")
        s = lax.axis_index("s")

        def zlp(j, car):
            def zin(k, car2):
                hbuf[j, pl.ds(k * 16, 16)] = jnp.zeros((16,), jnp.float32)
                return car2
            lax.fori_loop(0, half // 16, zin, 0)
            zbuf16[j, pl.ds(0, 16)] = jnp.zeros((16,), jnp.float32)
            ones16[j, pl.ds(0, 16)] = jnp.ones((16,), jnp.float32)
            return car
        lax.fori_loop(0, SGC, zlp, 0)

        def z2(j, car):
            pltpu.sync_copy(hbuf, acc_sh.at[pl.ds(s * rows_pt + j * SGC, SGC)])
            pltpu.sync_copy(zbuf16, cnt_sh.at[pl.ds(s * rows_pt + j * SGC, SGC)])
            return car
        lax.fori_loop(0, rows_pt // SGC, z2, 0)
        plsc.subcore_barrier()

        def slp(k, car):
            g = s + NS * k

            @pl.when(g < nch)
            def _():
                pltpu.sync_copy(ic.at[g], iv2.at[0])
                pltpu.sync_copy(
                    h_ref.at[pl.ds(g * SGC, SGC), pl.ds(c * half, half)], hbuf)
                pltpu.sync_copy(hbuf, acc_sh.at[iv2.at[0]], add=True)

                @pl.when(c == 0)
                def _():
                    pltpu.sync_copy(ones16, cnt_sh.at[iv2.at[0]], add=True)
            return car
        lax.fori_loop(0, kmax, slp, 0)
        plsc.subcore_barrier()

        pltpu.sync_copy(
            acc_sh.at[pl.ds(s * rows_pt, rows_pt)],
            ssum_o.at[pl.ds(s * rows_pt, rows_pt), pl.ds(c * half, half)])

        @pl.when(c == 0)
        def _():
            pltpu.sync_copy(cnt_sh.at[pl.ds(s * rows_pt, rows_pt)],
                            cnt_o.at[pl.ds(s * rows_pt, rows_pt)])

    return pl.kernel(
        body,
        out_type=[jax.ShapeDtypeStruct((NPAD, D), jnp.float32),
                  jax.ShapeDtypeStruct((NPAD, 16), jnp.float32)],
        mesh=_sc_mesh(),
        scratch_types=[pltpu.VMEM_SHARED((NPAD, half), jnp.float32),
                       pltpu.VMEM_SHARED((NPAD, 16), jnp.float32),
                       pltpu.VMEM((1, SGC), jnp.int32),
                       pltpu.VMEM((SGC, half), jnp.float32),
                       pltpu.VMEM((SGC, 16), jnp.float32),
                       pltpu.VMEM((SGC, 16), jnp.float32)],
    )(h, idx2)


def _sc_brow(batch, rowf):
    n = batch.shape[0]
    E = rowf.shape[0]
    per_w = E // NW
    pw_pad = (per_w + 15) // 16 * 16

    def body(b_ref, r_ref, o_ref, bt, iv, ov):
        w = lax.axis_index("s") * NC + lax.axis_index("c")
        pltpu.sync_copy(b_ref, bt)
        iv[pl.ds(pw_pad - 16, 16)] = jnp.zeros((16,), jnp.int32)
        pltpu.sync_copy(r_ref.at[pl.ds(w * per_w, per_w)], iv.at[pl.ds(0, per_w)])

        def lp(i, car):
            idxv = iv[pl.ds(i * 16, 16)]
            ov[pl.ds(i * 16, 16)] = plsc.load_gather(bt, [idxv])
            return car
        lax.fori_loop(0, pw_pad // 16, lp, 0)
        pltpu.sync_copy(ov.at[pl.ds(0, per_w)], o_ref.at[pl.ds(w * per_w, per_w)])

    return pl.kernel(
        body,
        out_type=jax.ShapeDtypeStruct((E,), jnp.int32),
        mesh=_sc_mesh(),
        compiler_params=pltpu.CompilerParams(needs_layout_passes=False),
        scratch_types=[pltpu.VMEM((n,), jnp.int32),
                       pltpu.VMEM((pw_pad,), jnp.int32),
                       pltpu.VMEM((pw_pad,), jnp.int32)],
    )(batch, rowf)


def _act_ln(h, g, be):
    h = jnp.where(h >= 0, h, 0.01 * h)
    mu = jnp.mean(h, axis=-1, keepdims=True)
    var = jnp.mean((h - mu) ** 2, axis=-1, keepdims=True)
    return (h - mu) * jax.lax.rsqrt(var + 1e-5) * g + be


def _dot(a, b):
    return jnp.dot(a, b, preferred_element_type=jnp.float32)


def _k1_body(has_u, xr_ref, xc_ref, ea_ref, brow_ref, ub_ref,
             wa, wb, wc, b1e, ge, bee, w2e, b2e,
             wna, wnb, b1n, gn, ben, w2n, b2n,
             e_ref, h_ref):
    xr = xr_ref[...]
    pre = _dot(xr, wa[...]) + _dot(xc_ref[...], wb[...]) + _dot(ea_ref[...], wc[...]) + b1e[...]
    if has_u:
        iot = jax.lax.broadcasted_iota(jnp.int32, (xr.shape[0], NG), 1)
        oh = (brow_ref[...] == iot).astype(jnp.float32)
        pre = pre + _dot(oh, ub_ref[...])
    e = _dot(_act_ln(pre, ge[...], bee[...]), w2e[...]) + b2e[...]
    e_ref[...] = e
    preh = _dot(xr, wna[...]) + _dot(e, wnb[...]) + b1n[...]
    h_ref[...] = _dot(_act_ln(preh, gn[...], ben[...]), w2n[...]) + b2n[...]


def _edge_node1(pe, pn, xr, xc, ea, brow, ub):
    E, dx = xr.shape
    de = ea.shape[1]
    H = pe["w2"].shape[0]
    has_u = ub is not None
    wa = pe["w1"][:dx]
    wb = pe["w1"][dx:2 * dx]
    wc = pe["w1"][2 * dx:2 * dx + de]
    wna = pn["w1"][:dx]
    wnb = pn["w1"][dx:dx + H]
    r1 = lambda a: a.reshape(1, -1)
    full = lambda shape: pl.BlockSpec(shape, lambda i: (0,) * len(shape))
    grid = (pl.cdiv(E, EB),)
    if not has_u:
        brow = jnp.zeros((E, 1), jnp.int32)
        ub = jnp.zeros((NG, H), jnp.float32)
    args = [xr, xc, ea, brow, ub,
            wa, wb, wc, r1(pe["b1"]), r1(pe["g"]), r1(pe["be"]), pe["w2"], r1(pe["b2"]),
            wna, wnb, r1(pn["b1"]), r1(pn["g"]), r1(pn["be"]), pn["w2"], r1(pn["b2"])]
    in_specs = [
        pl.BlockSpec((EB, dx), lambda i: (i, 0)),
        pl.BlockSpec((EB, dx), lambda i: (i, 0)),
        pl.BlockSpec((EB, de), lambda i: (i, 0)),
        pl.BlockSpec((EB, 1), lambda i: (i, 0)),
        full((NG, H)),
        full((dx, H)), full((dx, H)), full((de, H)), full((1, H)), full((1, H)),
        full((1, H)), full((H, H)), full((1, H)),
        full((dx, H)), full((H, H)), full((1, H)), full((1, H)), full((1, H)),
        full((H, H)), full((1, H)),
    ]
    return pl.pallas_call(
        functools.partial(_k1_body, has_u),
        grid=grid,
        in_specs=in_specs,
        out_specs=[pl.BlockSpec((EB, H), lambda i: (i, 0)),
                   pl.BlockSpec((EB, H), lambda i: (i, 0))],
        out_shape=[jax.ShapeDtypeStruct((E, H), jnp.float32),
                   jax.ShapeDtypeStruct((E, H), jnp.float32)],
    )(*args)


def _k3_body(has_u, x_ref, ssum_ref, cnt_ref, batch_ref, ub2_ref,
             wa, wb, b1, g, be, w2, b2,
             xn_ref, gsum_ref, gcnt_ref):
    xv = x_ref[...]
    agg = ssum_ref[...] * (1.0 / jnp.maximum(cnt_ref[...], 1.0))
    pre = _dot(xv, wa[...]) + _dot(agg, wb[...]) + b1[...]
    iot = jax.lax.broadcasted_iota(jnp.int32, (xv.shape[0], NG), 1)
    oh = (batch_ref[...] == iot).astype(jnp.float32)
    if has_u:
        pre = pre + _dot(oh, ub2_ref[...])
    xn = _dot(_act_ln(pre, g[...], be[...]), w2[...]) + b2[...]
    xn_ref[...] = xn
    dn = (((0,), (0,)), ((), ()))
    psum = jax.lax.dot_general(oh, xn, dn, preferred_element_type=jnp.float32)
    pcnt = jax.lax.dot_general(oh, jnp.ones_like(xn), dn, preferred_element_type=jnp.float32)

    @pl.when(pl.program_id(0) == 0)
    def _():
        gsum_ref[...] = jnp.zeros_like(gsum_ref)
        gcnt_ref[...] = jnp.zeros_like(gcnt_ref)

    gsum_ref[...] += psum
    gcnt_ref[...] += pcnt


def _node2(pn2, x_cur, ssum, cnt, batch_c, ub2):
    n, dx = x_cur.shape
    H = pn2["w2"].shape[0]
    has_u = ub2 is not None
    wa = pn2["w1"][:dx]
    wb = pn2["w1"][dx:dx + H]
    r1 = lambda a: a.reshape(1, -1)
    full = lambda shape: pl.BlockSpec(shape, lambda i: (0,) * len(shape))
    if not has_u:
        ub2 = jnp.zeros((NG, H), jnp.float32)
    grid = (pl.cdiv(n, NB),)
    args = [x_cur, ssum, cnt, batch_c, ub2,
            wa, wb, r1(pn2["b1"]), r1(pn2["g"]), r1(pn2["be"]), pn2["w2"], r1(pn2["b2"])]
    in_specs = [
        pl.BlockSpec((NB, dx), lambda i: (i, 0)),
        pl.BlockSpec((NB, H), lambda i: (i, 0)),
        pl.BlockSpec((NB, 1), lambda i: (i, 0)),
        pl.BlockSpec((NB, 1), lambda i: (i, 0)),
        full((NG, H)),
        full((dx, H)), full((H, H)), full((1, H)), full((1, H)), full((1, H)),
        full((H, H)), full((1, H)),
    ]
    return pl.pallas_call(
        functools.partial(_k3_body, has_u),
        grid=grid,
        in_specs=in_specs,
        out_specs=[pl.BlockSpec((NB, H), lambda i: (i, 0)),
                   full((NG, H)), full((NG, H))],
        out_shape=[jax.ShapeDtypeStruct((n, H), jnp.float32),
                   jax.ShapeDtypeStruct((NG, H), jnp.float32),
                   jax.ShapeDtypeStruct((NG, H), jnp.float32)],
    )(*args)


def _k4_body(has_u, u_ref, gsum_ref, gcnt_ref,
             wu, wm, b1, g, be, w2, b2, wde, wd2,
             uo_ref, ube_ref, ub2_ref):
    mean = gsum_ref[...] * (1.0 / jnp.maximum(gcnt_ref[...], 1.0))
    pre = _dot(mean, wm[...]) + b1[...]
    if has_u:
        pre = pre + _dot(u_ref[...], wu[...])
    uo = _dot(_act_ln(pre, g[...], be[...]), w2[...]) + b2[...]
    uo_ref[...] = uo
    ube_ref[...] = _dot(uo, wde[...])
    ub2_ref[...] = _dot(uo, wd2[...])


def _glob(pg, u, gsum, gcnt, wde, wd2):
    H = gsum.shape[1]
    GH = pg["w2"].shape[1]
    has_u = u is not None
    if has_u:
        wu = pg["w1"][:GH]
        wm = pg["w1"][GH:GH + H]
    else:
        u = jnp.zeros((NG, GH), jnp.float32)
        wu = jnp.zeros((GH, pg["w1"].shape[1]), jnp.float32)
        wm = pg["w1"]
    HH = wde.shape[1]
    r1 = lambda a: a.reshape(1, -1)
    return pl.pallas_call(
        functools.partial(_k4_body, has_u),
        out_shape=[jax.ShapeDtypeStruct((NG, GH), jnp.float32),
                   jax.ShapeDtypeStruct((NG, HH), jnp.float32),
                   jax.ShapeDtypeStruct((NG, HH), jnp.float32)],
    )(u, gsum, gcnt, wu, wm, r1(pg["b1"]), r1(pg["g"]), r1(pg["be"]),
      pg["w2"], r1(pg["b2"]), wde, wd2)


def kernel(x, edge_attr, params, edge_index, batch):
    row, col = edge_index[0], edge_index[1]
    n = x.shape[0]
    E = row.shape[0]
    H = 256
    batch_c = batch.reshape(n, 1)
    row2 = row.reshape(E // GC, GC)
    col2 = col.reshape(E // GC, GC)
    cols2 = col.reshape(E // SGC, SGC)
    brow = batch[row].reshape(E, 1)

    p = params["l1"]
    xr, xc = _sc_gather2(x, row2, col2)
    e, h = _edge_node1(p["edge"], p["node1"], xr, xc, edge_attr, None, None)
    ssum = jax.ops.segment_sum(h, col, num_segments=n)
    cnt = jax.ops.segment_sum(jnp.ones((E, 1), jnp.float32), col, num_segments=n)
    xn, gsum, gcnt = _node2(p["node2"], x, ssum, cnt, batch_c, None)
    pn = params["l2"]
    u, ube, ub2 = _glob(p["glob"], None, gsum, gcnt,
                        pn["edge"]["w1"][3 * H:], pn["node2"]["w1"][2 * H:])
    x_cur, ea = xn, e

    for name, nxt in (("l2", "l3"), ("l3", None)):
        p = params[name]
        xr, xc = _sc_gather2(x_cur, row2, col2)
        e, h = _edge_node1(p["edge"], p["node1"], xr, xc, ea, brow, ube)
        ssum = jax.ops.segment_sum(h, col, num_segments=n)
        xn, gsum, gcnt = _node2(p["node2"], x_cur, ssum, cnt, batch_c, ub2)
        if nxt is None:
            wde = jnp.zeros((params[name]["glob"]["w2"].shape[1], H), jnp.float32)
            wd2 = wde
        else:
            pn = params[nxt]
            wde = pn["edge"]["w1"][3 * H:]
            wd2 = pn["node2"]["w1"][2 * H:]
        u, ube, ub2 = _glob(p["glob"], u, gsum, gcnt, wde, wd2)
        x_cur, ea = xn, e
    return u


# speedup vs baseline: 2.9200x; 1.5659x over previous
import functools

import jax
import jax.numpy as jnp
from jax import lax
from jax.experimental import pallas as pl
from jax.experimental.pallas import tpu as pltpu
from jax.experimental.pallas import tpu_sc as plsc

NG = 64
EB = 640
NB = 400
NC, NS = 2, 16
NW = NC * NS
GC = 128
NPAD = 10240


def _sc_mesh():
    return plsc.VectorSubcoreMesh(
        core_axis_name="c", subcore_axis_name="s", num_cores=NC, num_subcores=NS)


def _sc_gather2(table, idxr2, idxc2):
    n, D = table.shape
    nch = idxr2.shape[0]
    E = nch * GC
    kmax = pl.cdiv(nch, NW)

    def body(tab, ir, ic, xr_o, xc_o, iv, buf, sem):
        w = lax.axis_index("s") * NC + lax.axis_index("c")

        def make_loop(isrc, out):
            def lp(k, car):
                g = w + NW * k

                @pl.when(g < nch)
                def _():
                    pltpu.sync_copy(isrc.at[g], iv)
                    pltpu.async_copy(tab.at[iv], buf, sem).wait()
                    pltpu.sync_copy(buf, out.at[pl.ds(g * GC, GC)])
                return car
            return lp

        lax.fori_loop(0, kmax, make_loop(ir, xr_o), 0)
        lax.fori_loop(0, kmax, make_loop(ic, xc_o), 0)

    return pl.kernel(
        body,
        out_type=[jax.ShapeDtypeStruct((E, D), jnp.float32),
                  jax.ShapeDtypeStruct((E, D), jnp.float32)],
        mesh=_sc_mesh(),
        scratch_types=[pltpu.VMEM((GC,), jnp.int32),
                       pltpu.VMEM((GC, D), jnp.float32),
                       pltpu.SemaphoreType.DMA],
    )(table, idxr2, idxc2)


SGC = 64


def _sc_scatter(h, idx2):
    E, D = h.shape
    half = D // NC
    rows_pt = NPAD // NS
    nch = idx2.shape[0]
    kmax = pl.cdiv(nch, NS)

    def body(h_ref, ic, ssum_o, acc_sh, iv2, hbuf):
        c = lax.axis_index("c")
        s = lax.axis_index("s")

        def zlp(j, car):
            def zin(k, car2):
                hbuf[j, pl.ds(k * 16, 16)] = jnp.zeros((16,), jnp.float32)
                return car2
            lax.fori_loop(0, half // 16, zin, 0)
            return car
        lax.fori_loop(0, SGC, zlp, 0)

        def z2(j, car):
            pltpu.sync_copy(hbuf, acc_sh.at[pl.ds(s * rows_pt + j * SGC, SGC)])
            return car
        lax.fori_loop(0, rows_pt // SGC, z2, 0)
        plsc.subcore_barrier()

        def slp(k, car):
            g = s + NS * k

            @pl.when(g < nch)
            def _():
                pltpu.sync_copy(ic.at[g], iv2.at[0])
                pltpu.sync_copy(
                    h_ref.at[pl.ds(g * SGC, SGC), pl.ds(c * half, half)], hbuf)
                pltpu.sync_copy(hbuf, acc_sh.at[iv2.at[0]], add=True)
            return car
        lax.fori_loop(0, kmax, slp, 0)
        plsc.subcore_barrier()

        pltpu.sync_copy(
            acc_sh.at[pl.ds(s * rows_pt, rows_pt)],
            ssum_o.at[pl.ds(s * rows_pt, rows_pt), pl.ds(c * half, half)])

    return pl.kernel(
        body,
        out_type=jax.ShapeDtypeStruct((NPAD, D), jnp.float32),
        mesh=_sc_mesh(),
        scratch_types=[pltpu.VMEM_SHARED((NPAD, half), jnp.float32),
                       pltpu.VMEM((1, SGC), jnp.int32),
                       pltpu.VMEM((SGC, half), jnp.float32)],
    )(h, idx2)


def _sc_counts(idx2):
    nch = idx2.shape[0]
    rows_pt = NPAD // NS
    kmax = pl.cdiv(nch, NS)

    def body(ic, cnt_o, cnt_sh, iv2, ones):
        c = lax.axis_index("c")
        s = lax.axis_index("s")

        def fill(val):
            def flp(j, car):
                def fin(k, car2):
                    ones[j, pl.ds(k * 16, 16)] = jnp.full((16,), val, jnp.float32)
                    return car2
                lax.fori_loop(0, 8, fin, 0)
                return car
            lax.fori_loop(0, SGC, flp, 0)

        fill(0.0)

        def z2(j, car):
            pltpu.sync_copy(ones, cnt_sh.at[pl.ds(s * rows_pt + j * SGC, SGC)])
            return car
        lax.fori_loop(0, rows_pt // SGC, z2, 0)
        fill(1.0)
        plsc.subcore_barrier()

        @pl.when(c == 0)
        def _():
            def slp(k, car):
                g = s + NS * k

                @pl.when(g < nch)
                def _():
                    pltpu.sync_copy(ic.at[g], iv2.at[0])
                    pltpu.sync_copy(ones, cnt_sh.at[iv2.at[0]], add=True)
                return car
            lax.fori_loop(0, kmax, slp, 0)
        plsc.subcore_barrier()

        @pl.when(c == 0)
        def _():
            pltpu.sync_copy(cnt_sh.at[pl.ds(s * rows_pt, rows_pt)],
                            cnt_o.at[pl.ds(s * rows_pt, rows_pt)])

    return pl.kernel(
        body,
        out_type=jax.ShapeDtypeStruct((NPAD, 128), jnp.float32),
        mesh=_sc_mesh(),
        scratch_types=[pltpu.VMEM_SHARED((NPAD, 128), jnp.float32),
                       pltpu.VMEM((1, SGC), jnp.int32),
                       pltpu.VMEM((SGC, 128), jnp.float32)],
    )(idx2)


def _sc_brow(batch, rowf):
    n = batch.shape[0]
    E = rowf.shape[0]
    per_w = E // NW
    pw_pad = (per_w + 15) // 16 * 16

    def body(b_ref, r_ref, o_ref, bt, iv, ov):
        w = lax.axis_index("s") * NC + lax.axis_index("c")
        pltpu.sync_copy(b_ref, bt)
        iv[pl.ds(pw_pad - 16, 16)] = jnp.zeros((16,), jnp.int32)
        pltpu.sync_copy(r_ref.at[pl.ds(w * per_w, per_w)], iv.at[pl.ds(0, per_w)])

        def lp(i, car):
            idxv = iv[pl.ds(i * 16, 16)]
            ov[pl.ds(i * 16, 16)] = plsc.load_gather(bt, [idxv])
            return car
        lax.fori_loop(0, pw_pad // 16, lp, 0)
        pltpu.sync_copy(ov.at[pl.ds(0, per_w)], o_ref.at[pl.ds(w * per_w, per_w)])

    return pl.kernel(
        body,
        out_type=jax.ShapeDtypeStruct((E,), jnp.int32),
        mesh=_sc_mesh(),
        compiler_params=pltpu.CompilerParams(needs_layout_passes=False),
        scratch_types=[pltpu.VMEM((n,), jnp.int32),
                       pltpu.VMEM((pw_pad,), jnp.int32),
                       pltpu.VMEM((pw_pad,), jnp.int32)],
    )(batch, rowf)


def _act_ln(h, g, be):
    h = jnp.where(h >= 0, h, 0.01 * h)
    mu = jnp.mean(h, axis=-1, keepdims=True)
    var = jnp.mean((h - mu) ** 2, axis=-1, keepdims=True)
    return (h - mu) * jax.lax.rsqrt(var + 1e-5) * g + be


def _dot(a, b):
    return jnp.dot(a, b, preferred_element_type=jnp.float32)


def _k1_body(has_u, xr_ref, xc_ref, ea_ref, brow_ref, ub_ref,
             wa, wb, wc, b1e, ge, bee, w2e, b2e,
             wna, wnb, b1n, gn, ben, w2n, b2n,
             e_ref, h_ref):
    xr = xr_ref[...]
    pre = _dot(xr, wa[...]) + _dot(xc_ref[...], wb[...]) + _dot(ea_ref[...], wc[...]) + b1e[...]
    if has_u:
        iot = jax.lax.broadcasted_iota(jnp.int32, (xr.shape[0], NG), 1)
        oh = (brow_ref[...] == iot).astype(jnp.float32)
        pre = pre + _dot(oh, ub_ref[...])
    e = _dot(_act_ln(pre, ge[...], bee[...]), w2e[...]) + b2e[...]
    e_ref[...] = e
    preh = _dot(xr, wna[...]) + _dot(e, wnb[...]) + b1n[...]
    h_ref[...] = _dot(_act_ln(preh, gn[...], ben[...]), w2n[...]) + b2n[...]


def _edge_node1(pe, pn, xr, xc, ea, brow, ub):
    E, dx = xr.shape
    de = ea.shape[1]
    H = pe["w2"].shape[0]
    has_u = ub is not None
    wa = pe["w1"][:dx]
    wb = pe["w1"][dx:2 * dx]
    wc = pe["w1"][2 * dx:2 * dx + de]
    wna = pn["w1"][:dx]
    wnb = pn["w1"][dx:dx + H]
    r1 = lambda a: a.reshape(1, -1)
    full = lambda shape: pl.BlockSpec(shape, lambda i: (0,) * len(shape))
    grid = (pl.cdiv(E, EB),)
    if not has_u:
        brow = jnp.zeros((E, 1), jnp.int32)
        ub = jnp.zeros((NG, H), jnp.float32)
    args = [xr, xc, ea, brow, ub,
            wa, wb, wc, r1(pe["b1"]), r1(pe["g"]), r1(pe["be"]), pe["w2"], r1(pe["b2"]),
            wna, wnb, r1(pn["b1"]), r1(pn["g"]), r1(pn["be"]), pn["w2"], r1(pn["b2"])]
    in_specs = [
        pl.BlockSpec((EB, dx), lambda i: (i, 0)),
        pl.BlockSpec((EB, dx), lambda i: (i, 0)),
        pl.BlockSpec((EB, de), lambda i: (i, 0)),
        pl.BlockSpec((EB, 1), lambda i: (i, 0)),
        full((NG, H)),
        full((dx, H)), full((dx, H)), full((de, H)), full((1, H)), full((1, H)),
        full((1, H)), full((H, H)), full((1, H)),
        full((dx, H)), full((H, H)), full((1, H)), full((1, H)), full((1, H)),
        full((H, H)), full((1, H)),
    ]
    return pl.pallas_call(
        functools.partial(_k1_body, has_u),
        grid=grid,
        in_specs=in_specs,
        out_specs=[pl.BlockSpec((EB, H), lambda i: (i, 0)),
                   pl.BlockSpec((EB, H), lambda i: (i, 0))],
        out_shape=[jax.ShapeDtypeStruct((E, H), jnp.float32),
                   jax.ShapeDtypeStruct((E, H), jnp.float32)],
    )(*args)


def _k3_body(has_u, x_ref, ssum_ref, cnt_ref, batch_ref, ub2_ref,
             wa, wb, b1, g, be, w2, b2,
             xn_ref, gsum_ref, gcnt_ref):
    xv = x_ref[...]
    agg = ssum_ref[...] * (1.0 / jnp.maximum(cnt_ref[...], 1.0))
    pre = _dot(xv, wa[...]) + _dot(agg, wb[...]) + b1[...]
    iot = jax.lax.broadcasted_iota(jnp.int32, (xv.shape[0], NG), 1)
    oh = (batch_ref[...] == iot).astype(jnp.float32)
    if has_u:
        pre = pre + _dot(oh, ub2_ref[...])
    xn = _dot(_act_ln(pre, g[...], be[...]), w2[...]) + b2[...]
    xn_ref[...] = xn
    dn = (((0,), (0,)), ((), ()))
    psum = jax.lax.dot_general(oh, xn, dn, preferred_element_type=jnp.float32)
    pcnt = jax.lax.dot_general(oh, jnp.ones_like(xn), dn, preferred_element_type=jnp.float32)

    @pl.when(pl.program_id(0) == 0)
    def _():
        gsum_ref[...] = jnp.zeros_like(gsum_ref)
        gcnt_ref[...] = jnp.zeros_like(gcnt_ref)

    gsum_ref[...] += psum
    gcnt_ref[...] += pcnt


def _node2(pn2, x_cur, ssum, cnt, batch_c, ub2):
    n, dx = x_cur.shape
    H = pn2["w2"].shape[0]
    has_u = ub2 is not None
    wa = pn2["w1"][:dx]
    wb = pn2["w1"][dx:dx + H]
    r1 = lambda a: a.reshape(1, -1)
    full = lambda shape: pl.BlockSpec(shape, lambda i: (0,) * len(shape))
    if not has_u:
        ub2 = jnp.zeros((NG, H), jnp.float32)
    grid = (pl.cdiv(n, NB),)
    args = [x_cur, ssum, cnt, batch_c, ub2,
            wa, wb, r1(pn2["b1"]), r1(pn2["g"]), r1(pn2["be"]), pn2["w2"], r1(pn2["b2"])]
    in_specs = [
        pl.BlockSpec((NB, dx), lambda i: (i, 0)),
        pl.BlockSpec((NB, H), lambda i: (i, 0)),
        pl.BlockSpec((NB, 1), lambda i: (i, 0)),
        pl.BlockSpec((NB, 1), lambda i: (i, 0)),
        full((NG, H)),
        full((dx, H)), full((H, H)), full((1, H)), full((1, H)), full((1, H)),
        full((H, H)), full((1, H)),
    ]
    return pl.pallas_call(
        functools.partial(_k3_body, has_u),
        grid=grid,
        in_specs=in_specs,
        out_specs=[pl.BlockSpec((NB, H), lambda i: (i, 0)),
                   full((NG, H)), full((NG, H))],
        out_shape=[jax.ShapeDtypeStruct((n, H), jnp.float32),
                   jax.ShapeDtypeStruct((NG, H), jnp.float32),
                   jax.ShapeDtypeStruct((NG, H), jnp.float32)],
    )(*args)


def _k4_body(has_u, u_ref, gsum_ref, gcnt_ref,
             wu, wm, b1, g, be, w2, b2, wde, wd2,
             uo_ref, ube_ref, ub2_ref):
    mean = gsum_ref[...] * (1.0 / jnp.maximum(gcnt_ref[...], 1.0))
    pre = _dot(mean, wm[...]) + b1[...]
    if has_u:
        pre = pre + _dot(u_ref[...], wu[...])
    uo = _dot(_act_ln(pre, g[...], be[...]), w2[...]) + b2[...]
    uo_ref[...] = uo
    ube_ref[...] = _dot(uo, wde[...])
    ub2_ref[...] = _dot(uo, wd2[...])


def _glob(pg, u, gsum, gcnt, wde, wd2):
    H = gsum.shape[1]
    GH = pg["w2"].shape[1]
    has_u = u is not None
    if has_u:
        wu = pg["w1"][:GH]
        wm = pg["w1"][GH:GH + H]
    else:
        u = jnp.zeros((NG, GH), jnp.float32)
        wu = jnp.zeros((GH, pg["w1"].shape[1]), jnp.float32)
        wm = pg["w1"]
    HH = wde.shape[1]
    r1 = lambda a: a.reshape(1, -1)
    return pl.pallas_call(
        functools.partial(_k4_body, has_u),
        out_shape=[jax.ShapeDtypeStruct((NG, GH), jnp.float32),
                   jax.ShapeDtypeStruct((NG, HH), jnp.float32),
                   jax.ShapeDtypeStruct((NG, HH), jnp.float32)],
    )(u, gsum, gcnt, wu, wm, r1(pg["b1"]), r1(pg["g"]), r1(pg["be"]),
      pg["w2"], r1(pg["b2"]), wde, wd2)


def kernel(x, edge_attr, params, edge_index, batch):
    row, col = edge_index[0], edge_index[1]
    n = x.shape[0]
    E = row.shape[0]
    H = 256
    batch_c = batch.reshape(n, 1)
    row2 = row.reshape(E // GC, GC)
    col2 = col.reshape(E // GC, GC)
    cols2 = col.reshape(E // SGC, SGC)
    brow = _sc_brow(batch, row).reshape(E, 1)

    p = params["l1"]
    xr, xc = _sc_gather2(x, row2, col2)
    e, h = _edge_node1(p["edge"], p["node1"], xr, xc, edge_attr, None, None)
    ssum = _sc_scatter(h, cols2)
    cnt = _sc_counts(cols2)[:, :1]
    xn, gsum, gcnt = _node2(p["node2"], x, ssum, cnt, batch_c, None)
    pn = params["l2"]
    u, ube, ub2 = _glob(p["glob"], None, gsum, gcnt,
                        pn["edge"]["w1"][3 * H:], pn["node2"]["w1"][2 * H:])
    x_cur, ea = xn, e

    for name, nxt in (("l2", "l3"), ("l3", None)):
        p = params[name]
        xr, xc = _sc_gather2(x_cur, row2, col2)
        e, h = _edge_node1(p["edge"], p["node1"], xr, xc, ea, brow, ube)
        ssum = _sc_scatter(h, cols2)
        xn, gsum, gcnt = _node2(p["node2"], x_cur, ssum, cnt, batch_c, ub2)
        if nxt is None:
            wde = jnp.zeros((params[name]["glob"]["w2"].shape[1], H), jnp.float32)
            wd2 = wde
        else:
            pn = params[nxt]
            wde = pn["edge"]["w1"][3 * H:]
            wd2 = pn["node2"]["w1"][2 * H:]
        u, ube, ub2 = _glob(p["glob"], u, gsum, gcnt, wde, wd2)
        x_cur, ea = xn, e
    return u


# speedup vs baseline: 3.1356x; 1.0738x over previous
import functools

import jax
import jax.numpy as jnp
from jax import lax
from jax.experimental import pallas as pl
from jax.experimental.pallas import tpu as pltpu
from jax.experimental.pallas import tpu_sc as plsc

NG = 64
EB = 640
NB = 400
NC, NS = 2, 16
NW = NC * NS
GC = 128
NPAD = 10240


def _sc_mesh():
    return plsc.VectorSubcoreMesh(
        core_axis_name="c", subcore_axis_name="s", num_cores=NC, num_subcores=NS)


def _sc_gather2(table, idxr3, idxc3):
    n, D = table.shape
    kmax = idxr3.shape[1]
    nch = 160000 // GC
    E = 160000

    def body(tab, ir, ic, xr_o, xc_o, iv, buf0, buf1, buf2, s0, s1, s2, o0, o1, o2):
        w = lax.axis_index("s") * NC + lax.axis_index("c")
        bufs = (buf0, buf1, buf2)
        gsems = (s0, s1, s2)
        osems = (o0, o1, o2)

        def run(isrc, out):
            pltpu.sync_copy(isrc.at[w], iv)

            def active(k):
                return (k >= 0) & (k < kmax) & (w + NW * k < nch)

            def start_gather(k, b):
                @pl.when(active(k))
                def _():
                    pltpu.async_copy(tab.at[iv.at[k]], bufs[b], gsems[b])

            def finish(k, b):
                @pl.when(active(k))
                def _():
                    g = w + NW * k
                    pltpu.make_async_copy(tab.at[iv.at[k]], bufs[b], gsems[b]).wait()
                    pltpu.async_copy(bufs[b], out.at[pl.ds(g * GC, GC)], osems[b])

            def drain(k, b):
                @pl.when(active(k))
                def _():
                    g = w + NW * k
                    pltpu.make_async_copy(
                        bufs[b], out.at[pl.ds(g * GC, GC)], osems[b]).wait()

            start_gather(0, 0)
            start_gather(1, 1)

            def lp(k3, car):
                for b in range(3):
                    k = 3 * k3 + b
                    drain(k - 1, (b + 2) % 3)
                    start_gather(k + 2, (b + 2) % 3)
                    finish(k, b)
                return car
            lax.fori_loop(0, kmax // 3, lp, 0)
            drain(kmax - 1, (kmax - 1) % 3)

        run(ir, xr_o)
        run(ic, xc_o)

    return pl.kernel(
        body,
        out_type=[jax.ShapeDtypeStruct((E, D), jnp.float32),
                  jax.ShapeDtypeStruct((E, D), jnp.float32)],
        mesh=_sc_mesh(),
        scratch_types=[pltpu.VMEM((kmax, GC), jnp.int32),
                       pltpu.VMEM((GC, D), jnp.float32),
                       pltpu.VMEM((GC, D), jnp.float32),
                       pltpu.VMEM((GC, D), jnp.float32),
                       pltpu.SemaphoreType.DMA,
                       pltpu.SemaphoreType.DMA,
                       pltpu.SemaphoreType.DMA,
                       pltpu.SemaphoreType.DMA,
                       pltpu.SemaphoreType.DMA,
                       pltpu.SemaphoreType.DMA],
    )(table, idxr3, idxc3)


SGC = 64


def _sc_scatter(h, idx2):
    E, D = h.shape
    half = D // NC
    rows_pt = NPAD // NS
    nch = idx2.shape[0]
    kmax = pl.cdiv(nch, NS)

    def body(h_ref, ic, ssum_o, acc_sh, iv2, hbuf):
        c = lax.axis_index("c")
        s = lax.axis_index("s")

        def zlp(j, car):
            def zin(k, car2):
                hbuf[j, pl.ds(k * 16, 16)] = jnp.zeros((16,), jnp.float32)
                return car2
            lax.fori_loop(0, half // 16, zin, 0)
            return car
        lax.fori_loop(0, SGC, zlp, 0)

        def z2(j, car):
            pltpu.sync_copy(hbuf, acc_sh.at[pl.ds(s * rows_pt + j * SGC, SGC)])
            return car
        lax.fori_loop(0, rows_pt // SGC, z2, 0)
        plsc.subcore_barrier()

        def slp(k, car):
            g = s + NS * k

            @pl.when(g < nch)
            def _():
                pltpu.sync_copy(ic.at[g], iv2.at[0])
                pltpu.sync_copy(
                    h_ref.at[pl.ds(g * SGC, SGC), pl.ds(c * half, half)], hbuf)
                pltpu.sync_copy(hbuf, acc_sh.at[iv2.at[0]], add=True)
            return car
        lax.fori_loop(0, kmax, slp, 0)
        plsc.subcore_barrier()

        pltpu.sync_copy(
            acc_sh.at[pl.ds(s * rows_pt, rows_pt)],
            ssum_o.at[pl.ds(s * rows_pt, rows_pt), pl.ds(c * half, half)])

    return pl.kernel(
        body,
        out_type=jax.ShapeDtypeStruct((NPAD, D), jnp.float32),
        mesh=_sc_mesh(),
        scratch_types=[pltpu.VMEM_SHARED((NPAD, half), jnp.float32),
                       pltpu.VMEM((1, SGC), jnp.int32),
                       pltpu.VMEM((SGC, half), jnp.float32)],
    )(h, idx2)


def _sc_counts(idx2):
    nch = idx2.shape[0]
    rows_pt = NPAD // NS
    kmax = pl.cdiv(nch, NS)

    def body(ic, cnt_o, cnt_sh, iv2, ones):
        c = lax.axis_index("c")
        s = lax.axis_index("s")

        def fill(val):
            def flp(j, car):
                def fin(k, car2):
                    ones[j, pl.ds(k * 16, 16)] = jnp.full((16,), val, jnp.float32)
                    return car2
                lax.fori_loop(0, 8, fin, 0)
                return car
            lax.fori_loop(0, SGC, flp, 0)

        fill(0.0)

        def z2(j, car):
            pltpu.sync_copy(ones, cnt_sh.at[pl.ds(s * rows_pt + j * SGC, SGC)])
            return car
        lax.fori_loop(0, rows_pt // SGC, z2, 0)
        fill(1.0)
        plsc.subcore_barrier()

        @pl.when(c == 0)
        def _():
            def slp(k, car):
                g = s + NS * k

                @pl.when(g < nch)
                def _():
                    pltpu.sync_copy(ic.at[g], iv2.at[0])
                    pltpu.sync_copy(ones, cnt_sh.at[iv2.at[0]], add=True)
                return car
            lax.fori_loop(0, kmax, slp, 0)
        plsc.subcore_barrier()

        @pl.when(c == 0)
        def _():
            pltpu.sync_copy(cnt_sh.at[pl.ds(s * rows_pt, rows_pt)],
                            cnt_o.at[pl.ds(s * rows_pt, rows_pt)])

    return pl.kernel(
        body,
        out_type=jax.ShapeDtypeStruct((NPAD, 128), jnp.float32),
        mesh=_sc_mesh(),
        scratch_types=[pltpu.VMEM_SHARED((NPAD, 128), jnp.float32),
                       pltpu.VMEM((1, SGC), jnp.int32),
                       pltpu.VMEM((SGC, 128), jnp.float32)],
    )(idx2)


def _sc_brow(batch, rowf):
    n = batch.shape[0]
    E = rowf.shape[0]
    per_w = E // NW
    pw_pad = (per_w + 15) // 16 * 16

    def body(b_ref, r_ref, o_ref, bt, iv, ov):
        w = lax.axis_index("s") * NC + lax.axis_index("c")
        pltpu.sync_copy(b_ref, bt)
        iv[pl.ds(pw_pad - 16, 16)] = jnp.zeros((16,), jnp.int32)
        pltpu.sync_copy(r_ref.at[pl.ds(w * per_w, per_w)], iv.at[pl.ds(0, per_w)])

        def lp(i, car):
            idxv = iv[pl.ds(i * 16, 16)]
            ov[pl.ds(i * 16, 16)] = plsc.load_gather(bt, [idxv])
            return car
        lax.fori_loop(0, pw_pad // 16, lp, 0)
        pltpu.sync_copy(ov.at[pl.ds(0, per_w)], o_ref.at[pl.ds(w * per_w, per_w)])

    return pl.kernel(
        body,
        out_type=jax.ShapeDtypeStruct((E,), jnp.int32),
        mesh=_sc_mesh(),
        compiler_params=pltpu.CompilerParams(needs_layout_passes=False),
        scratch_types=[pltpu.VMEM((n,), jnp.int32),
                       pltpu.VMEM((pw_pad,), jnp.int32),
                       pltpu.VMEM((pw_pad,), jnp.int32)],
    )(batch, rowf)


def _act_ln(h, g, be):
    h = jnp.where(h >= 0, h, 0.01 * h)
    mu = jnp.mean(h, axis=-1, keepdims=True)
    var = jnp.mean((h - mu) ** 2, axis=-1, keepdims=True)
    return (h - mu) * jax.lax.rsqrt(var + 1e-5) * g + be


def _dot(a, b):
    return jnp.dot(a, b, preferred_element_type=jnp.float32)


def _k1_body(has_u, xr_ref, xc_ref, ea_ref, brow_ref, ub_ref,
             wa, wb, wc, b1e, ge, bee, w2e, b2e,
             wna, wnb, b1n, gn, ben, w2n, b2n,
             e_ref, h_ref):
    xr = xr_ref[...]
    pre = _dot(xr, wa[...]) + _dot(xc_ref[...], wb[...]) + _dot(ea_ref[...], wc[...]) + b1e[...]
    if has_u:
        iot = jax.lax.broadcasted_iota(jnp.int32, (xr.shape[0], NG), 1)
        oh = (brow_ref[...] == iot).astype(jnp.float32)
        pre = pre + _dot(oh, ub_ref[...])
    e = _dot(_act_ln(pre, ge[...], bee[...]), w2e[...]) + b2e[...]
    e_ref[...] = e
    preh = _dot(xr, wna[...]) + _dot(e, wnb[...]) + b1n[...]
    h_ref[...] = _dot(_act_ln(preh, gn[...], ben[...]), w2n[...]) + b2n[...]


def _edge_node1(pe, pn, xr, xc, ea, brow, ub):
    E, dx = xr.shape
    de = ea.shape[1]
    H = pe["w2"].shape[0]
    has_u = ub is not None
    wa = pe["w1"][:dx]
    wb = pe["w1"][dx:2 * dx]
    wc = pe["w1"][2 * dx:2 * dx + de]
    wna = pn["w1"][:dx]
    wnb = pn["w1"][dx:dx + H]
    r1 = lambda a: a.reshape(1, -1)
    full = lambda shape: pl.BlockSpec(shape, lambda i: (0,) * len(shape))
    grid = (pl.cdiv(E, EB),)
    if not has_u:
        brow = jnp.zeros((E, 1), jnp.int32)
        ub = jnp.zeros((NG, H), jnp.float32)
    args = [xr, xc, ea, brow, ub,
            wa, wb, wc, r1(pe["b1"]), r1(pe["g"]), r1(pe["be"]), pe["w2"], r1(pe["b2"]),
            wna, wnb, r1(pn["b1"]), r1(pn["g"]), r1(pn["be"]), pn["w2"], r1(pn["b2"])]
    in_specs = [
        pl.BlockSpec((EB, dx), lambda i: (i, 0)),
        pl.BlockSpec((EB, dx), lambda i: (i, 0)),
        pl.BlockSpec((EB, de), lambda i: (i, 0)),
        pl.BlockSpec((EB, 1), lambda i: (i, 0)),
        full((NG, H)),
        full((dx, H)), full((dx, H)), full((de, H)), full((1, H)), full((1, H)),
        full((1, H)), full((H, H)), full((1, H)),
        full((dx, H)), full((H, H)), full((1, H)), full((1, H)), full((1, H)),
        full((H, H)), full((1, H)),
    ]
    return pl.pallas_call(
        functools.partial(_k1_body, has_u),
        grid=grid,
        in_specs=in_specs,
        out_specs=[pl.BlockSpec((EB, H), lambda i: (i, 0)),
                   pl.BlockSpec((EB, H), lambda i: (i, 0))],
        out_shape=[jax.ShapeDtypeStruct((E, H), jnp.float32),
                   jax.ShapeDtypeStruct((E, H), jnp.float32)],
    )(*args)


def _k3_body(has_u, x_ref, ssum_ref, cnt_ref, batch_ref, ub2_ref,
             wa, wb, b1, g, be, w2, b2,
             xn_ref, gsum_ref, gcnt_ref):
    xv = x_ref[...]
    agg = ssum_ref[...] * (1.0 / jnp.maximum(cnt_ref[...], 1.0))
    pre = _dot(xv, wa[...]) + _dot(agg, wb[...]) + b1[...]
    iot = jax.lax.broadcasted_iota(jnp.int32, (xv.shape[0], NG), 1)
    oh = (batch_ref[...] == iot).astype(jnp.float32)
    if has_u:
        pre = pre + _dot(oh, ub2_ref[...])
    xn = _dot(_act_ln(pre, g[...], be[...]), w2[...]) + b2[...]
    xn_ref[...] = xn
    dn = (((0,), (0,)), ((), ()))
    psum = jax.lax.dot_general(oh, xn, dn, preferred_element_type=jnp.float32)
    pcnt = jax.lax.dot_general(oh, jnp.ones_like(xn), dn, preferred_element_type=jnp.float32)

    @pl.when(pl.program_id(0) == 0)
    def _():
        gsum_ref[...] = jnp.zeros_like(gsum_ref)
        gcnt_ref[...] = jnp.zeros_like(gcnt_ref)

    gsum_ref[...] += psum
    gcnt_ref[...] += pcnt


def _node2(pn2, x_cur, ssum, cnt, batch_c, ub2):
    n, dx = x_cur.shape
    H = pn2["w2"].shape[0]
    has_u = ub2 is not None
    wa = pn2["w1"][:dx]
    wb = pn2["w1"][dx:dx + H]
    r1 = lambda a: a.reshape(1, -1)
    full = lambda shape: pl.BlockSpec(shape, lambda i: (0,) * len(shape))
    if not has_u:
        ub2 = jnp.zeros((NG, H), jnp.float32)
    grid = (pl.cdiv(n, NB),)
    args = [x_cur, ssum, cnt, batch_c, ub2,
            wa, wb, r1(pn2["b1"]), r1(pn2["g"]), r1(pn2["be"]), pn2["w2"], r1(pn2["b2"])]
    in_specs = [
        pl.BlockSpec((NB, dx), lambda i: (i, 0)),
        pl.BlockSpec((NB, H), lambda i: (i, 0)),
        pl.BlockSpec((NB, 1), lambda i: (i, 0)),
        pl.BlockSpec((NB, 1), lambda i: (i, 0)),
        full((NG, H)),
        full((dx, H)), full((H, H)), full((1, H)), full((1, H)), full((1, H)),
        full((H, H)), full((1, H)),
    ]
    return pl.pallas_call(
        functools.partial(_k3_body, has_u),
        grid=grid,
        in_specs=in_specs,
        out_specs=[pl.BlockSpec((NB, H), lambda i: (i, 0)),
                   full((NG, H)), full((NG, H))],
        out_shape=[jax.ShapeDtypeStruct((n, H), jnp.float32),
                   jax.ShapeDtypeStruct((NG, H), jnp.float32),
                   jax.ShapeDtypeStruct((NG, H), jnp.float32)],
    )(*args)


def _k4_body(has_u, u_ref, gsum_ref, gcnt_ref,
             wu, wm, b1, g, be, w2, b2, wde, wd2,
             uo_ref, ube_ref, ub2_ref):
    mean = gsum_ref[...] * (1.0 / jnp.maximum(gcnt_ref[...], 1.0))
    pre = _dot(mean, wm[...]) + b1[...]
    if has_u:
        pre = pre + _dot(u_ref[...], wu[...])
    uo = _dot(_act_ln(pre, g[...], be[...]), w2[...]) + b2[...]
    uo_ref[...] = uo
    ube_ref[...] = _dot(uo, wde[...])
    ub2_ref[...] = _dot(uo, wd2[...])


def _glob(pg, u, gsum, gcnt, wde, wd2):
    H = gsum.shape[1]
    GH = pg["w2"].shape[1]
    has_u = u is not None
    if has_u:
        wu = pg["w1"][:GH]
        wm = pg["w1"][GH:GH + H]
    else:
        u = jnp.zeros((NG, GH), jnp.float32)
        wu = jnp.zeros((GH, pg["w1"].shape[1]), jnp.float32)
        wm = pg["w1"]
    HH = wde.shape[1]
    r1 = lambda a: a.reshape(1, -1)
    return pl.pallas_call(
        functools.partial(_k4_body, has_u),
        out_shape=[jax.ShapeDtypeStruct((NG, GH), jnp.float32),
                   jax.ShapeDtypeStruct((NG, HH), jnp.float32),
                   jax.ShapeDtypeStruct((NG, HH), jnp.float32)],
    )(u, gsum, gcnt, wu, wm, r1(pg["b1"]), r1(pg["g"]), r1(pg["be"]),
      pg["w2"], r1(pg["b2"]), wde, wd2)


def kernel(x, edge_attr, params, edge_index, batch):
    row, col = edge_index[0], edge_index[1]
    n = x.shape[0]
    E = row.shape[0]
    H = 256
    batch_c = batch.reshape(n, 1)
    def _bulk(idx):
        i2 = idx.reshape(E // GC, GC)
        pad = jnp.zeros(((NW * 42 - E // GC), GC), jnp.int32)
        return jnp.concatenate([i2, pad], 0).reshape(42, NW, GC).transpose(1, 0, 2)
    row3 = _bulk(row)
    col3 = _bulk(col)
    cols2 = col.reshape(E // SGC, SGC)
    brow = _sc_brow(batch, row).reshape(E, 1)

    p = params["l1"]
    xr, xc = _sc_gather2(x, row3, col3)
    e, h = _edge_node1(p["edge"], p["node1"], xr, xc, edge_attr, None, None)
    ssum = _sc_scatter(h, cols2)
    cnt = _sc_counts(cols2)[:, :1]
    xn, gsum, gcnt = _node2(p["node2"], x, ssum, cnt, batch_c, None)
    pn = params["l2"]
    u, ube, ub2 = _glob(p["glob"], None, gsum, gcnt,
                        pn["edge"]["w1"][3 * H:], pn["node2"]["w1"][2 * H:])
    x_cur, ea = xn, e

    for name, nxt in (("l2", "l3"), ("l3", None)):
        p = params[name]
        xr, xc = _sc_gather2(x_cur, row3, col3)
        e, h = _edge_node1(p["edge"], p["node1"], xr, xc, ea, brow, ube)
        ssum = _sc_scatter(h, cols2)
        xn, gsum, gcnt = _node2(p["node2"], x_cur, ssum, cnt, batch_c, ub2)
        if nxt is None:
            wde = jnp.zeros((params[name]["glob"]["w2"].shape[1], H), jnp.float32)
            wd2 = wde
        else:
            pn = params[nxt]
            wde = pn["edge"]["w1"][3 * H:]
            wd2 = pn["node2"]["w1"][2 * H:]
        u, ube, ub2 = _glob(p["glob"], u, gsum, gcnt, wde, wd2)
        x_cur, ea = xn, e
    return u


# speedup vs baseline: 3.5933x; 1.1460x over previous
import functools

import jax
import jax.numpy as jnp
from jax import lax
from jax.experimental import pallas as pl
from jax.experimental.pallas import tpu as pltpu
from jax.experimental.pallas import tpu_sc as plsc

NG = 64
EB = 640
NB = 400
NC, NS = 2, 16
NW = NC * NS
GC = 128
NPAD = 10240


def _sc_mesh():
    return plsc.VectorSubcoreMesh(
        core_axis_name="c", subcore_axis_name="s", num_cores=NC, num_subcores=NS)


def _sc_gather2(table, idxr3, idxc3, batch=None, rowf=None):
    n, D = table.shape
    kmax = idxr3.shape[1]
    nch = 160000 // GC
    E = 160000
    per_w = E // NW
    pw_pad = (per_w + 15) // 16 * 16
    with_brow = batch is not None

    def body(*refs):
        if with_brow:
            (tab, ir, ic, b_ref, r_ref, xr_o, xc_o, brow_o,
             iv, buf0, buf1, buf2, s0, s1, s2, o0, o1, o2, bt, ivb, ovb) = refs
        else:
            (tab, ir, ic, xr_o, xc_o,
             iv, buf0, buf1, buf2, s0, s1, s2, o0, o1, o2) = refs
        w = lax.axis_index("s") * NC + lax.axis_index("c")
        bufs = (buf0, buf1, buf2)
        gsems = (s0, s1, s2)
        osems = (o0, o1, o2)

        def run(isrc, out):
            pltpu.sync_copy(isrc.at[w], iv)

            def active(k):
                return (k >= 0) & (k < kmax) & (w + NW * k < nch)

            def start_gather(k, b):
                @pl.when(active(k))
                def _():
                    pltpu.async_copy(tab.at[iv.at[k]], bufs[b], gsems[b])

            def finish(k, b):
                @pl.when(active(k))
                def _():
                    g = w + NW * k
                    pltpu.make_async_copy(tab.at[iv.at[k]], bufs[b], gsems[b]).wait()
                    pltpu.async_copy(bufs[b], out.at[pl.ds(g * GC, GC)], osems[b])

            def drain(k, b):
                @pl.when(active(k))
                def _():
                    g = w + NW * k
                    pltpu.make_async_copy(
                        bufs[b], out.at[pl.ds(g * GC, GC)], osems[b]).wait()

            start_gather(0, 0)
            start_gather(1, 1)

            def lp(k3, car):
                for b in range(3):
                    k = 3 * k3 + b
                    drain(k - 1, (b + 2) % 3)
                    start_gather(k + 2, (b + 2) % 3)
                    finish(k, b)
                return car
            lax.fori_loop(0, kmax // 3, lp, 0)
            drain(kmax - 1, (kmax - 1) % 3)

        if with_brow:
            pltpu.sync_copy(b_ref, bt)
            ivb[pl.ds(pw_pad - 16, 16)] = jnp.zeros((16,), jnp.int32)
            pltpu.sync_copy(r_ref.at[pl.ds(w * per_w, per_w)],
                            ivb.at[pl.ds(0, per_w)])

            def blp(i, car):
                idxv = ivb[pl.ds(i * 16, 16)]
                ovb[pl.ds(i * 16, 16)] = plsc.load_gather(bt, [idxv])
                return car
            lax.fori_loop(0, pw_pad // 16, blp, 0)
            pltpu.sync_copy(ovb.at[pl.ds(0, per_w)],
                            brow_o.at[pl.ds(w * per_w, per_w)])

        run(ir, xr_o)
        run(ic, xc_o)

    outs = [jax.ShapeDtypeStruct((E, D), jnp.float32),
            jax.ShapeDtypeStruct((E, D), jnp.float32)]
    scratch = [pltpu.VMEM((kmax, GC), jnp.int32),
               pltpu.VMEM((GC, D), jnp.float32),
               pltpu.VMEM((GC, D), jnp.float32),
               pltpu.VMEM((GC, D), jnp.float32),
               pltpu.SemaphoreType.DMA,
               pltpu.SemaphoreType.DMA,
               pltpu.SemaphoreType.DMA,
               pltpu.SemaphoreType.DMA,
               pltpu.SemaphoreType.DMA,
               pltpu.SemaphoreType.DMA]
    args = [table, idxr3, idxc3]
    cp = None
    if with_brow:
        outs = outs + [jax.ShapeDtypeStruct((E,), jnp.int32)]
        scratch = scratch + [pltpu.VMEM((n,), jnp.int32),
                             pltpu.VMEM((pw_pad,), jnp.int32),
                             pltpu.VMEM((pw_pad,), jnp.int32)]
        args = [table, idxr3, idxc3, batch, rowf]
        cp = pltpu.CompilerParams(needs_layout_passes=False)
    return pl.kernel(
        body,
        out_type=outs,
        mesh=_sc_mesh(),
        compiler_params=cp,
        scratch_types=scratch,
    )(*args)


SGC = 64


def _sc_scatter(h, idx2):
    E, D = h.shape
    half = D // NC
    rows_pt = NPAD // NS
    nch = idx2.shape[0]
    kmax = pl.cdiv(nch, NS)

    def body(h_ref, ic, ssum_o, acc_sh, iv2, hbuf0, hbuf1, hs0, hs1):
        c = lax.axis_index("c")
        s = lax.axis_index("s")
        hbufs = (hbuf0, hbuf1)
        hsems = (hs0, hs1)

        def zlp(j, car):
            def zin(k, car2):
                hbuf0[j, pl.ds(k * 16, 16)] = jnp.zeros((16,), jnp.float32)
                return car2
            lax.fori_loop(0, half // 16, zin, 0)
            return car
        lax.fori_loop(0, SGC, zlp, 0)

        def z2(j, car):
            pltpu.sync_copy(hbuf0, acc_sh.at[pl.ds(s * rows_pt + j * SGC, SGC)])
            return car
        lax.fori_loop(0, rows_pt // SGC, z2, 0)
        plsc.subcore_barrier()

        def active(k):
            return (k >= 0) & (s + NS * k < nch)

        def hread(k, b):
            @pl.when(active(k))
            def _():
                g = s + NS * k
                pltpu.async_copy(
                    h_ref.at[pl.ds(g * SGC, SGC), pl.ds(c * half, half)],
                    hbufs[b], hsems[b])

        hread(0, 0)

        def slp(k2, car):
            for b in range(2):
                k = 2 * k2 + b
                hread(k + 1, 1 - b)

                @pl.when(active(k))
                def _():
                    g = s + NS * k
                    pltpu.make_async_copy(
                        h_ref.at[pl.ds(g * SGC, SGC), pl.ds(c * half, half)],
                        hbufs[b], hsems[b]).wait()
                    pltpu.sync_copy(ic.at[g], iv2.at[0])
                    pltpu.sync_copy(hbufs[b], acc_sh.at[iv2.at[0]], add=True)
            return car
        lax.fori_loop(0, (kmax + 1) // 2, slp, 0)
        plsc.subcore_barrier()

        pltpu.sync_copy(
            acc_sh.at[pl.ds(s * rows_pt, rows_pt)],
            ssum_o.at[pl.ds(s * rows_pt, rows_pt), pl.ds(c * half, half)])

    return pl.kernel(
        body,
        out_type=jax.ShapeDtypeStruct((NPAD, D), jnp.float32),
        mesh=_sc_mesh(),
        scratch_types=[pltpu.VMEM_SHARED((NPAD, half), jnp.float32),
                       pltpu.VMEM((1, SGC), jnp.int32),
                       pltpu.VMEM((SGC, half), jnp.float32),
                       pltpu.VMEM((SGC, half), jnp.float32),
                       pltpu.SemaphoreType.DMA,
                       pltpu.SemaphoreType.DMA],
    )(h, idx2)


def _sc_counts(idx2):
    nch = idx2.shape[0]
    rows_pt = NPAD // NS
    kmax = pl.cdiv(nch, NS)

    def body(ic, cnt_o, cnt_sh, iv2, ones):
        c = lax.axis_index("c")
        s = lax.axis_index("s")

        def fill(val):
            def flp(j, car):
                def fin(k, car2):
                    ones[j, pl.ds(k * 16, 16)] = jnp.full((16,), val, jnp.float32)
                    return car2
                lax.fori_loop(0, 8, fin, 0)
                return car
            lax.fori_loop(0, SGC, flp, 0)

        fill(0.0)

        def z2(j, car):
            pltpu.sync_copy(ones, cnt_sh.at[pl.ds(s * rows_pt + j * SGC, SGC)])
            return car
        lax.fori_loop(0, rows_pt // SGC, z2, 0)
        fill(1.0)
        plsc.subcore_barrier()

        @pl.when(c == 0)
        def _():
            def slp(k, car):
                g = s + NS * k

                @pl.when(g < nch)
                def _():
                    pltpu.sync_copy(ic.at[g], iv2.at[0])
                    pltpu.sync_copy(ones, cnt_sh.at[iv2.at[0]], add=True)
                return car
            lax.fori_loop(0, kmax, slp, 0)
        plsc.subcore_barrier()

        @pl.when(c == 0)
        def _():
            pltpu.sync_copy(cnt_sh.at[pl.ds(s * rows_pt, rows_pt)],
                            cnt_o.at[pl.ds(s * rows_pt, rows_pt)])

    return pl.kernel(
        body,
        out_type=jax.ShapeDtypeStruct((NPAD, 128), jnp.float32),
        mesh=_sc_mesh(),
        scratch_types=[pltpu.VMEM_SHARED((NPAD, 128), jnp.float32),
                       pltpu.VMEM((1, SGC), jnp.int32),
                       pltpu.VMEM((SGC, 128), jnp.float32)],
    )(idx2)


def _sc_brow(batch, rowf):
    n = batch.shape[0]
    E = rowf.shape[0]
    per_w = E // NW
    pw_pad = (per_w + 15) // 16 * 16

    def body(b_ref, r_ref, o_ref, bt, iv, ov):
        w = lax.axis_index("s") * NC + lax.axis_index("c")
        pltpu.sync_copy(b_ref, bt)
        iv[pl.ds(pw_pad - 16, 16)] = jnp.zeros((16,), jnp.int32)
        pltpu.sync_copy(r_ref.at[pl.ds(w * per_w, per_w)], iv.at[pl.ds(0, per_w)])

        def lp(i, car):
            idxv = iv[pl.ds(i * 16, 16)]
            ov[pl.ds(i * 16, 16)] = plsc.load_gather(bt, [idxv])
            return car
        lax.fori_loop(0, pw_pad // 16, lp, 0)
        pltpu.sync_copy(ov.at[pl.ds(0, per_w)], o_ref.at[pl.ds(w * per_w, per_w)])

    return pl.kernel(
        body,
        out_type=jax.ShapeDtypeStruct((E,), jnp.int32),
        mesh=_sc_mesh(),
        compiler_params=pltpu.CompilerParams(needs_layout_passes=False),
        scratch_types=[pltpu.VMEM((n,), jnp.int32),
                       pltpu.VMEM((pw_pad,), jnp.int32),
                       pltpu.VMEM((pw_pad,), jnp.int32)],
    )(batch, rowf)


def _act_ln(h, g, be):
    h = jnp.where(h >= 0, h, 0.01 * h)
    mu = jnp.mean(h, axis=-1, keepdims=True)
    var = jnp.mean((h - mu) ** 2, axis=-1, keepdims=True)
    return (h - mu) * jax.lax.rsqrt(var + 1e-5) * g + be


def _dot(a, b):
    return jnp.dot(a, b, preferred_element_type=jnp.float32)


def _k1_body(has_u, xr_ref, xc_ref, ea_ref, brow_ref, ub_ref,
             wa, wb, wc, b1e, ge, bee, w2e, b2e,
             wna, wnb, b1n, gn, ben, w2n, b2n,
             e_ref, h_ref):
    xr = xr_ref[...]
    pre = _dot(xr, wa[...]) + _dot(xc_ref[...], wb[...]) + _dot(ea_ref[...], wc[...]) + b1e[...]
    if has_u:
        iot = jax.lax.broadcasted_iota(jnp.int32, (xr.shape[0], NG), 1)
        oh = (brow_ref[...] == iot).astype(jnp.float32)
        pre = pre + _dot(oh, ub_ref[...])
    e = _dot(_act_ln(pre, ge[...], bee[...]), w2e[...]) + b2e[...]
    e_ref[...] = e
    preh = _dot(xr, wna[...]) + _dot(e, wnb[...]) + b1n[...]
    h_ref[...] = _dot(_act_ln(preh, gn[...], ben[...]), w2n[...]) + b2n[...]


def _edge_node1(pe, pn, xr, xc, ea, brow, ub):
    E, dx = xr.shape
    de = ea.shape[1]
    H = pe["w2"].shape[0]
    has_u = ub is not None
    wa = pe["w1"][:dx]
    wb = pe["w1"][dx:2 * dx]
    wc = pe["w1"][2 * dx:2 * dx + de]
    wna = pn["w1"][:dx]
    wnb = pn["w1"][dx:dx + H]
    r1 = lambda a: a.reshape(1, -1)
    full = lambda shape: pl.BlockSpec(shape, lambda i: (0,) * len(shape))
    grid = (pl.cdiv(E, EB),)
    if not has_u:
        brow = jnp.zeros((E, 1), jnp.int32)
        ub = jnp.zeros((NG, H), jnp.float32)
    args = [xr, xc, ea, brow, ub,
            wa, wb, wc, r1(pe["b1"]), r1(pe["g"]), r1(pe["be"]), pe["w2"], r1(pe["b2"]),
            wna, wnb, r1(pn["b1"]), r1(pn["g"]), r1(pn["be"]), pn["w2"], r1(pn["b2"])]
    in_specs = [
        pl.BlockSpec((EB, dx), lambda i: (i, 0)),
        pl.BlockSpec((EB, dx), lambda i: (i, 0)),
        pl.BlockSpec((EB, de), lambda i: (i, 0)),
        pl.BlockSpec((EB, 1), lambda i: (i, 0)),
        full((NG, H)),
        full((dx, H)), full((dx, H)), full((de, H)), full((1, H)), full((1, H)),
        full((1, H)), full((H, H)), full((1, H)),
        full((dx, H)), full((H, H)), full((1, H)), full((1, H)), full((1, H)),
        full((H, H)), full((1, H)),
    ]
    return pl.pallas_call(
        functools.partial(_k1_body, has_u),
        grid=grid,
        in_specs=in_specs,
        out_specs=[pl.BlockSpec((EB, H), lambda i: (i, 0)),
                   pl.BlockSpec((EB, H), lambda i: (i, 0))],
        out_shape=[jax.ShapeDtypeStruct((E, H), jnp.float32),
                   jax.ShapeDtypeStruct((E, H), jnp.float32)],
    )(*args)


def _k3_body(has_u, x_ref, ssum_ref, cnt_ref, batch_ref, ub2_ref,
             wa, wb, b1, g, be, w2, b2,
             xn_ref, gsum_ref, gcnt_ref):
    xv = x_ref[...]
    agg = ssum_ref[...] * (1.0 / jnp.maximum(cnt_ref[...], 1.0))
    pre = _dot(xv, wa[...]) + _dot(agg, wb[...]) + b1[...]
    iot = jax.lax.broadcasted_iota(jnp.int32, (xv.shape[0], NG), 1)
    oh = (batch_ref[...] == iot).astype(jnp.float32)
    if has_u:
        pre = pre + _dot(oh, ub2_ref[...])
    xn = _dot(_act_ln(pre, g[...], be[...]), w2[...]) + b2[...]
    xn_ref[...] = xn
    dn = (((0,), (0,)), ((), ()))
    psum = jax.lax.dot_general(oh, xn, dn, preferred_element_type=jnp.float32)
    pcnt = jax.lax.dot_general(oh, jnp.ones_like(xn), dn, preferred_element_type=jnp.float32)

    @pl.when(pl.program_id(0) == 0)
    def _():
        gsum_ref[...] = jnp.zeros_like(gsum_ref)
        gcnt_ref[...] = jnp.zeros_like(gcnt_ref)

    gsum_ref[...] += psum
    gcnt_ref[...] += pcnt


def _node2(pn2, x_cur, ssum, cnt, batch_c, ub2):
    n, dx = x_cur.shape
    H = pn2["w2"].shape[0]
    has_u = ub2 is not None
    wa = pn2["w1"][:dx]
    wb = pn2["w1"][dx:dx + H]
    r1 = lambda a: a.reshape(1, -1)
    full = lambda shape: pl.BlockSpec(shape, lambda i: (0,) * len(shape))
    if not has_u:
        ub2 = jnp.zeros((NG, H), jnp.float32)
    grid = (pl.cdiv(n, NB),)
    args = [x_cur, ssum, cnt, batch_c, ub2,
            wa, wb, r1(pn2["b1"]), r1(pn2["g"]), r1(pn2["be"]), pn2["w2"], r1(pn2["b2"])]
    in_specs = [
        pl.BlockSpec((NB, dx), lambda i: (i, 0)),
        pl.BlockSpec((NB, H), lambda i: (i, 0)),
        pl.BlockSpec((NB, 1), lambda i: (i, 0)),
        pl.BlockSpec((NB, 1), lambda i: (i, 0)),
        full((NG, H)),
        full((dx, H)), full((H, H)), full((1, H)), full((1, H)), full((1, H)),
        full((H, H)), full((1, H)),
    ]
    return pl.pallas_call(
        functools.partial(_k3_body, has_u),
        grid=grid,
        in_specs=in_specs,
        out_specs=[pl.BlockSpec((NB, H), lambda i: (i, 0)),
                   full((NG, H)), full((NG, H))],
        out_shape=[jax.ShapeDtypeStruct((n, H), jnp.float32),
                   jax.ShapeDtypeStruct((NG, H), jnp.float32),
                   jax.ShapeDtypeStruct((NG, H), jnp.float32)],
    )(*args)


def _k4_body(has_u, u_ref, gsum_ref, gcnt_ref,
             wu, wm, b1, g, be, w2, b2, wde, wd2,
             uo_ref, ube_ref, ub2_ref):
    mean = gsum_ref[...] * (1.0 / jnp.maximum(gcnt_ref[...], 1.0))
    pre = _dot(mean, wm[...]) + b1[...]
    if has_u:
        pre = pre + _dot(u_ref[...], wu[...])
    uo = _dot(_act_ln(pre, g[...], be[...]), w2[...]) + b2[...]
    uo_ref[...] = uo
    ube_ref[...] = _dot(uo, wde[...])
    ub2_ref[...] = _dot(uo, wd2[...])


def _glob(pg, u, gsum, gcnt, wde, wd2):
    H = gsum.shape[1]
    GH = pg["w2"].shape[1]
    has_u = u is not None
    if has_u:
        wu = pg["w1"][:GH]
        wm = pg["w1"][GH:GH + H]
    else:
        u = jnp.zeros((NG, GH), jnp.float32)
        wu = jnp.zeros((GH, pg["w1"].shape[1]), jnp.float32)
        wm = pg["w1"]
    HH = wde.shape[1]
    r1 = lambda a: a.reshape(1, -1)
    return pl.pallas_call(
        functools.partial(_k4_body, has_u),
        out_shape=[jax.ShapeDtypeStruct((NG, GH), jnp.float32),
                   jax.ShapeDtypeStruct((NG, HH), jnp.float32),
                   jax.ShapeDtypeStruct((NG, HH), jnp.float32)],
    )(u, gsum, gcnt, wu, wm, r1(pg["b1"]), r1(pg["g"]), r1(pg["be"]),
      pg["w2"], r1(pg["b2"]), wde, wd2)


def kernel(x, edge_attr, params, edge_index, batch):
    row, col = edge_index[0], edge_index[1]
    n = x.shape[0]
    E = row.shape[0]
    H = 256
    batch_c = batch.reshape(n, 1)
    def _bulk(idx):
        i2 = idx.reshape(E // GC, GC)
        pad = jnp.zeros(((NW * 42 - E // GC), GC), jnp.int32)
        return jnp.concatenate([i2, pad], 0).reshape(42, NW, GC).transpose(1, 0, 2)
    row3 = _bulk(row)
    col3 = _bulk(col)
    cols2 = col.reshape(E // SGC, SGC)


    p = params["l1"]
    xr, xc, browf = _sc_gather2(x, row3, col3, batch, row)
    brow = browf.reshape(E, 1)
    e, h = _edge_node1(p["edge"], p["node1"], xr, xc, edge_attr, None, None)
    ssum = _sc_scatter(h, cols2)
    cnt = _sc_counts(cols2)[:, :1]
    xn, gsum, gcnt = _node2(p["node2"], x, ssum, cnt, batch_c, None)
    pn = params["l2"]
    u, ube, ub2 = _glob(p["glob"], None, gsum, gcnt,
                        pn["edge"]["w1"][3 * H:], pn["node2"]["w1"][2 * H:])
    x_cur, ea = xn, e

    for name, nxt in (("l2", "l3"), ("l3", None)):
        p = params[name]
        xr, xc = _sc_gather2(x_cur, row3, col3)
        e, h = _edge_node1(p["edge"], p["node1"], xr, xc, ea, brow, ube)
        ssum = _sc_scatter(h, cols2)
        xn, gsum, gcnt = _node2(p["node2"], x_cur, ssum, cnt, batch_c, ub2)
        if nxt is None:
            wde = jnp.zeros((params[name]["glob"]["w2"].shape[1], H), jnp.float32)
            wd2 = wde
        else:
            pn = params[nxt]
            wde = pn["edge"]["w1"][3 * H:]
            wd2 = pn["node2"]["w1"][2 * H:]
        u, ube, ub2 = _glob(p["glob"], u, gsum, gcnt, wde, wd2)
        x_cur, ea = xn, e
    return u
